# baseline (device time: 176720 ns/iter reference)
import functools
from types import SimpleNamespace

import jax
import jax.numpy as jnp
from jax import lax
from jax.experimental import pallas as pl
from jax.experimental.pallas import tpu as pltpu

N_DEV = 8
SQ = 512
HSQ = 256
SKV = 2048
D = 1024
HQ = 8
DH = 128
SCALE = 0.08838834764831843
BF16 = jnp.bfloat16
F32 = jnp.float32
MESH = pl.DeviceIdType.MESH


def kernel(x, Wq, Wo, K_ext, V_ext):
    x2 = x.reshape(SQ, D)
    K2 = K_ext.reshape(SKV, D)
    V2 = V_ext.reshape(SKV, D)

    def body(x_ref, wq_ref, wo_ref, k_ref, v_ref, out_ref,
             kbf, vbf,
             qc_cw, ac_cw, lc_cw, pv_cw, lt_cw,
             qc_ccw, ac_ccw, lc_ccw, pv_ccw, lt_ccw,
             qs_cw, qr_cw, as_cw, ar_cw, ls_cw, lr_cw,
             qs_ccw, qr_ccw, as_ccw, ar_ccw, ls_ccw, lr_ccw,
             qcred_cw, acred_cw, qcred_ccw, acred_ccw):
        my = lax.axis_index("i")
        left = (my - 1) % N_DEV
        right = (my + 1) % N_DEV

        barrier_sem = pltpu.get_barrier_semaphore()
        for nbr in [left, right]:
            pl.semaphore_signal(barrier_sem, inc=1, device_id=(nbr,),
                                device_id_type=MESH)
        pl.semaphore_wait(barrier_sem, 2)

        kbf[...] = k_ref[...].astype(BF16)
        vbf[...] = v_ref[...].astype(BF16)

        cw = SimpleNamespace(qc=qc_cw, ac=ac_cw, lc=lc_cw, pv=pv_cw,
                             lt=lt_cw, qs=qs_cw, qr=qr_cw, asd=as_cw,
                             ar=ar_cw, lsd=ls_cw, lr=lr_cw, qcred=qcred_cw,
                             acred=acred_cw, fwd=right, back=left, row0=0)
        ccw = SimpleNamespace(qc=qc_ccw, ac=ac_ccw, lc=lc_ccw, pv=pv_ccw,
                              lt=lt_ccw, qs=qs_ccw, qr=qr_ccw, asd=as_ccw,
                              ar=ar_ccw, lsd=ls_ccw, lr=lr_ccw,
                              qcred=qcred_ccw, acred=acred_ccw, fwd=left,
                              back=right, row0=HSQ)
        dirs = (cw, ccw)

        def rdma(src, dst, ssem, rsem, target):
            return pltpu.make_async_remote_copy(
                src_ref=src, dst_ref=dst, send_sem=ssem, recv_sem=rsem,
                device_id=(target,), device_id_type=MESH)

        def q_rdma(d, a, b):
            return rdma(d.qc.at[a], d.qc.at[b], d.qs.at[a], d.qr.at[b], d.fwd)

        def a_rdma(d, a, b):
            return rdma(d.ac.at[a], d.ac.at[b], d.asd.at[a], d.ar.at[b],
                        d.fwd)

        def l_rdma(d, a, b):
            return rdma(d.lc.at[a], d.lc.at[b], d.lsd.at[a], d.lr.at[b],
                        d.fwd)

        def compute_pv(qslot, acc_out, l_out):
            for head in range(HQ):
                c0 = head * DH
                qh = qslot[:, c0:c0 + DH]
                s = lax.dot_general(qh, kbf[:, c0:c0 + DH],
                                    (((1,), (1,)), ((), ())),
                                    preferred_element_type=F32)
                p = jnp.exp(s)
                l_out[:, head:head + 1] = jnp.sum(p, axis=1, keepdims=True)
                acc_out[:, c0:c0 + DH] = lax.dot_general(
                    p.astype(BF16), vbf[:, c0:c0 + DH],
                    (((1,), (0,)), ((), ())),
                    preferred_element_type=F32).astype(acc_out.dtype)

        wqbf = wq_ref[...].astype(BF16)
        for d in dirs:
            d.qc[0] = (lax.dot_general(
                x_ref[d.row0:d.row0 + HSQ, :].astype(BF16), wqbf,
                (((1,), (0,)), ((), ())),
                preferred_element_type=F32) * SCALE).astype(BF16)
            q_rdma(d, 0, 1).start()
        for d in dirs:
            compute_pv(d.qc.at[0], d.ac.at[0], d.lc.at[0])
            a_rdma(d, 0, 1).start()
            l_rdma(d, 0, 1).start()

        def round_body(t, carry):
            slot = lax.rem(t, 2)
            nslot = lax.rem(t + 1, 2)
            pslot = nslot
            for d in dirs:
                q_rdma(d, pslot, pslot).wait_send()
                pl.semaphore_signal(d.qcred, inc=1, device_id=(d.back,),
                                    device_id_type=MESH)
                a_rdma(d, pslot, pslot).wait_send()
                l_rdma(d, pslot, pslot).wait_send()
                pl.semaphore_signal(d.acred, inc=1, device_id=(d.back,),
                                    device_id_type=MESH)
            for d in dirs:
                q_rdma(d, slot, slot).wait_recv()
                pl.semaphore_wait(d.qcred, 1)
                q_rdma(d, slot, nslot).start()
            for d in dirs:
                compute_pv(d.qc.at[slot], d.pv, d.lt)
            for d in dirs:
                a_rdma(d, slot, slot).wait_recv()
                l_rdma(d, slot, slot).wait_recv()
                d.ac[slot] = (d.ac[slot].astype(F32)
                              + d.pv[...]).astype(BF16)
                d.lc[slot] = d.lc[slot] + d.lt[...]
                pl.semaphore_wait(d.acred, 1)
                a_rdma(d, slot, nslot).start()
                l_rdma(d, slot, nslot).start()
            return carry

        lax.fori_loop(1, 7, round_body, 0)

        for d in dirs:
            q_rdma(d, 0, 0).wait_send()
            a_rdma(d, 0, 0).wait_send()
            l_rdma(d, 0, 0).wait_send()
            pl.semaphore_signal(d.acred, inc=1, device_id=(d.back,),
                                device_id_type=MESH)
        for d in dirs:
            q_rdma(d, 1, 1).wait_recv()
            compute_pv(d.qc.at[1], d.pv, d.lt)
            a_rdma(d, 1, 1).wait_recv()
            l_rdma(d, 1, 1).wait_recv()
            d.ac[1] = (d.ac[1].astype(F32) + d.pv[...]).astype(BF16)
            d.lc[1] = d.lc[1] + d.lt[...]
            pl.semaphore_wait(d.acred, 1)
            a_rdma(d, 1, 0).start()
            l_rdma(d, 1, 0).start()

        for d in dirs:
            a_rdma(d, 0, 0).wait_recv()
            l_rdma(d, 0, 0).wait_recv()
            a_rdma(d, 1, 1).wait_send()
            l_rdma(d, 1, 1).wait_send()

        wobf = wo_ref[...].astype(BF16)
        for d in dirs:
            for head in range(HQ):
                c0 = head * DH
                d.ac[0, :, c0:c0 + DH] = (
                    d.ac[0, :, c0:c0 + DH].astype(F32)
                    / d.lc[0, :, head:head + 1]).astype(BF16)
            out_ref[d.row0:d.row0 + HSQ, :] = lax.dot_general(
                d.ac[0], wobf, (((1,), (0,)), ((), ())),
                preferred_element_type=F32)

        @functools.partial(pl.run_scoped,
                           second_barrier=pltpu.SemaphoreType.REGULAR)
        def _(second_barrier):
            for nbr in [left, right]:
                pl.semaphore_signal(second_barrier, inc=1, device_id=(nbr,),
                                    device_id_type=MESH)
            pl.semaphore_wait(second_barrier, 2)

    dma2 = pltpu.SemaphoreType.DMA((2,))
    dir_bufs = [
        pltpu.VMEM((2, HSQ, D), BF16),
        pltpu.VMEM((2, HSQ, D), BF16),
        pltpu.VMEM((2, HSQ, HQ), F32),
        pltpu.VMEM((HSQ, D), F32),
        pltpu.VMEM((HSQ, HQ), F32),
    ]
    out = pl.pallas_call(
        body,
        out_shape=jax.ShapeDtypeStruct((SQ, D), F32),
        in_specs=[pl.BlockSpec(memory_space=pltpu.VMEM)] * 5,
        out_specs=pl.BlockSpec(memory_space=pltpu.VMEM),
        scratch_shapes=(
            [pltpu.VMEM((SKV, D), BF16), pltpu.VMEM((SKV, D), BF16)]
            + dir_bufs + dir_bufs
            + [dma2] * 12
            + [pltpu.SemaphoreType.REGULAR] * 4
        ),
        compiler_params=pltpu.CompilerParams(collective_id=0),
    )(x2, Wq, Wo, K2, V2)
    return out.reshape(1, SQ, D)


# device time: 144356 ns/iter; 1.2242x vs baseline; 1.2242x over previous
import functools
from types import SimpleNamespace

import jax
import jax.numpy as jnp
from jax import lax
from jax.experimental import pallas as pl
from jax.experimental.pallas import tpu as pltpu

N_DEV = 8
SQ = 512
HSQ = 256
SKV = 2048
D = 1024
HQ = 8
DH = 128
SCALE = 0.08838834764831843
BF16 = jnp.bfloat16
F32 = jnp.float32
MESH = pl.DeviceIdType.MESH


def kernel(x, Wq, Wo, K_ext, V_ext):
    x2 = x.reshape(SQ, D)
    K2 = K_ext.reshape(SKV, D)
    V2 = V_ext.reshape(SKV, D)

    def body(x_ref, wq_ref, wo_ref, k_ref, v_ref, out_ref,
             kbf, vbf,
             qc_cw, ac_cw, lc_cw, pv_cw, lt_cw,
             qc_ccw, ac_ccw, lc_ccw, pv_ccw, lt_ccw,
             qs_cw, qr_cw, as_cw, ar_cw, ls_cw, lr_cw,
             qs_ccw, qr_ccw, as_ccw, ar_ccw, ls_ccw, lr_ccw,
             qcred_cw, acred_cw, qcred_ccw, acred_ccw):
        my = lax.axis_index("i")
        left = (my - 1) % N_DEV
        right = (my + 1) % N_DEV

        barrier_sem = pltpu.get_barrier_semaphore()
        for nbr in [left, right]:
            pl.semaphore_signal(barrier_sem, inc=1, device_id=(nbr,),
                                device_id_type=MESH)
        pl.semaphore_wait(barrier_sem, 2)

        kbf[...] = k_ref[...].astype(BF16)
        vbf[...] = v_ref[...].astype(BF16)

        cw = SimpleNamespace(qc=qc_cw, ac=ac_cw, lc=lc_cw, pv=pv_cw,
                             lt=lt_cw, qs=qs_cw, qr=qr_cw, asd=as_cw,
                             ar=ar_cw, lsd=ls_cw, lr=lr_cw, qcred=qcred_cw,
                             acred=acred_cw, fwd=right, back=left, row0=0)
        ccw = SimpleNamespace(qc=qc_ccw, ac=ac_ccw, lc=lc_ccw, pv=pv_ccw,
                              lt=lt_ccw, qs=qs_ccw, qr=qr_ccw, asd=as_ccw,
                              ar=ar_ccw, lsd=ls_ccw, lr=lr_ccw,
                              qcred=qcred_ccw, acred=acred_ccw, fwd=left,
                              back=right, row0=HSQ)
        dirs = (cw, ccw)

        def rdma(src, dst, ssem, rsem, target):
            return pltpu.make_async_remote_copy(
                src_ref=src, dst_ref=dst, send_sem=ssem, recv_sem=rsem,
                device_id=(target,), device_id_type=MESH)

        def q_rdma(d, a, b):
            return rdma(d.qc.at[a], d.qc.at[b], d.qs.at[a], d.qr.at[b], d.fwd)

        def a_rdma(d, a, b):
            return rdma(d.ac.at[a], d.ac.at[b], d.asd.at[a], d.ar.at[b],
                        d.fwd)

        def l_rdma(d, a, b):
            return rdma(d.lc.at[a], d.lc.at[b], d.lsd.at[a], d.lr.at[b],
                        d.fwd)

        def compute_pv(qslot, acc_out, l_out):
            for head in range(HQ):
                c0 = head * DH
                qh = qslot[:, c0:c0 + DH]
                s = lax.dot_general(qh, kbf[:, c0:c0 + DH],
                                    (((1,), (1,)), ((), ())),
                                    preferred_element_type=F32)
                p = jnp.exp(s)
                l_out[:, head:head + 1] = jnp.sum(p, axis=1, keepdims=True)
                acc_out[:, c0:c0 + DH] = lax.dot_general(
                    p.astype(BF16), vbf[:, c0:c0 + DH],
                    (((1,), (0,)), ((), ())),
                    preferred_element_type=F32).astype(acc_out.dtype)

        wqbf = wq_ref[...].astype(BF16)
        for d in dirs:
            d.qc[0] = (lax.dot_general(
                x_ref[d.row0:d.row0 + HSQ, :].astype(BF16), wqbf,
                (((1,), (0,)), ((), ())),
                preferred_element_type=F32) * SCALE).astype(BF16)
            q_rdma(d, 0, 1).start()
        for d in dirs:
            compute_pv(d.qc.at[0], d.ac.at[0], d.lc.at[0])
            a_rdma(d, 0, 1).start()
            l_rdma(d, 0, 1).start()

        def round_body(t, carry):
            slot = lax.rem(t, 2)
            nslot = lax.rem(t + 1, 2)
            pslot = nslot
            for d in dirs:
                q_rdma(d, slot, slot).wait_recv()
                q_rdma(d, pslot, pslot).wait_send()
                pl.semaphore_signal(d.qcred, inc=1, device_id=(d.back,),
                                    device_id_type=MESH)
                pl.semaphore_wait(d.qcred, 1)
                q_rdma(d, slot, nslot).start()
            for d in dirs:
                compute_pv(d.qc.at[slot], d.pv, d.lt)
            for d in dirs:
                a_rdma(d, pslot, pslot).wait_send()
                l_rdma(d, pslot, pslot).wait_send()
                pl.semaphore_signal(d.acred, inc=1, device_id=(d.back,),
                                    device_id_type=MESH)
            for d in dirs:
                a_rdma(d, slot, slot).wait_recv()
                l_rdma(d, slot, slot).wait_recv()
                d.ac[slot] = (d.ac[slot].astype(F32)
                              + d.pv[...]).astype(BF16)
                d.lc[slot] = d.lc[slot] + d.lt[...]
                pl.semaphore_wait(d.acred, 1)
                a_rdma(d, slot, nslot).start()
                l_rdma(d, slot, nslot).start()
            return carry

        lax.fori_loop(1, 7, round_body, 0)

        for d in dirs:
            q_rdma(d, 1, 1).wait_recv()
            q_rdma(d, 0, 0).wait_send()
            compute_pv(d.qc.at[1], d.pv, d.lt)
        for d in dirs:
            a_rdma(d, 0, 0).wait_send()
            l_rdma(d, 0, 0).wait_send()
            pl.semaphore_signal(d.acred, inc=1, device_id=(d.back,),
                                device_id_type=MESH)
        for d in dirs:
            a_rdma(d, 1, 1).wait_recv()
            l_rdma(d, 1, 1).wait_recv()
            d.ac[1] = (d.ac[1].astype(F32) + d.pv[...]).astype(BF16)
            d.lc[1] = d.lc[1] + d.lt[...]
            pl.semaphore_wait(d.acred, 1)
            a_rdma(d, 1, 0).start()
            l_rdma(d, 1, 0).start()

        for d in dirs:
            a_rdma(d, 0, 0).wait_recv()
            l_rdma(d, 0, 0).wait_recv()
            a_rdma(d, 1, 1).wait_send()
            l_rdma(d, 1, 1).wait_send()

        wobf = wo_ref[...].astype(BF16)
        for d in dirs:
            for head in range(HQ):
                c0 = head * DH
                d.ac[0, :, c0:c0 + DH] = (
                    d.ac[0, :, c0:c0 + DH].astype(F32)
                    / d.lc[0, :, head:head + 1]).astype(BF16)
            out_ref[d.row0:d.row0 + HSQ, :] = lax.dot_general(
                d.ac[0], wobf, (((1,), (0,)), ((), ())),
                preferred_element_type=F32)

        @functools.partial(pl.run_scoped,
                           second_barrier=pltpu.SemaphoreType.REGULAR)
        def _(second_barrier):
            for nbr in [left, right]:
                pl.semaphore_signal(second_barrier, inc=1, device_id=(nbr,),
                                    device_id_type=MESH)
            pl.semaphore_wait(second_barrier, 2)

    dma2 = pltpu.SemaphoreType.DMA((2,))
    dir_bufs = [
        pltpu.VMEM((2, HSQ, D), BF16),
        pltpu.VMEM((2, HSQ, D), BF16),
        pltpu.VMEM((2, HSQ, HQ), F32),
        pltpu.VMEM((HSQ, D), F32),
        pltpu.VMEM((HSQ, HQ), F32),
    ]
    out = pl.pallas_call(
        body,
        out_shape=jax.ShapeDtypeStruct((SQ, D), F32),
        in_specs=[pl.BlockSpec(memory_space=pltpu.VMEM)] * 5,
        out_specs=pl.BlockSpec(memory_space=pltpu.VMEM),
        scratch_shapes=(
            [pltpu.VMEM((SKV, D), BF16), pltpu.VMEM((SKV, D), BF16)]
            + dir_bufs + dir_bufs
            + [dma2] * 12
            + [pltpu.SemaphoreType.REGULAR] * 4
        ),
        compiler_params=pltpu.CompilerParams(collective_id=0),
    )(x2, Wq, Wo, K2, V2)
    return out.reshape(1, SQ, D)
